# speedup vs baseline: 4.3825x; 4.3825x over previous
import functools

import jax
import jax.numpy as jnp
from jax import lax
from jax.experimental import pallas as pl
from jax.experimental.pallas import tpu as pltpu
from jax.experimental.pallas import tpu_sc as plsc

N_NODES = 10000
HIDDEN = 256
HALF = 128
N_EDGES = 160000

NS = 16
CH = 128
CPT = -(-N_EDGES // (NS * CH))
EPT = CPT * CH
E_PAD = EPT * NS
N_JUNK = 8
U_ROWS = N_NODES + N_JUNK
ROWS_PER_TEC = 632
LAST_ROWS = U_ROWS - (NS - 1) * ROWS_PER_TEC
LAST_OUT_ROWS = N_NODES - (NS - 1) * ROWS_PER_TEC


def _sc_segment_sum(x_st, src, dst, zrows):
  mesh = plsc.VectorSubcoreMesh(core_axis_name="c", subcore_axis_name="s")

  @functools.partial(
      pl.kernel,
      out_type=jax.ShapeDtypeStruct((2, N_NODES, HALF), jnp.float32),
      mesh=mesh,
      scratch_types=[
          pltpu.VMEM((CH,), jnp.int32),
          pltpu.VMEM((CH,), jnp.int32),
          pltpu.VMEM((CH, HALF), jnp.float32),
          pltpu.VMEM_SHARED((U_ROWS, HALF), jnp.float32),
      ],
  )
  def seg_sum(x_hbm, src_hbm, dst_hbm, z_hbm, out_hbm,
              src_v, dst_v, rows_v, acc_sh):
    c = lax.axis_index("c")
    s = lax.axis_index("s")

    row0 = s * ROWS_PER_TEC

    @pl.when(s < NS - 1)
    def _():
      pltpu.sync_copy(z_hbm, acc_sh.at[pl.ds(row0, ROWS_PER_TEC)])

    @pl.when(s == NS - 1)
    def _():
      pltpu.sync_copy(z_hbm.at[pl.ds(0, LAST_ROWS)],
                      acc_sh.at[pl.ds((NS - 1) * ROWS_PER_TEC, LAST_ROWS)])

    plsc.subcore_barrier()

    base = s * EPT

    @pl.loop(0, CPT)
    def _(i):
      off = base + i * CH
      pltpu.sync_copy(src_hbm.at[pl.ds(off, CH)], src_v)
      pltpu.sync_copy(dst_hbm.at[pl.ds(off, CH)], dst_v)
      pltpu.sync_copy(x_hbm.at[c].at[src_v], rows_v)
      pltpu.sync_copy(rows_v, acc_sh.at[dst_v], add=True)

    plsc.subcore_barrier()

    @pl.when(s < NS - 1)
    def _():
      pltpu.sync_copy(acc_sh.at[pl.ds(row0, ROWS_PER_TEC)],
                      out_hbm.at[c].at[pl.ds(row0, ROWS_PER_TEC)])

    @pl.when(s == NS - 1)
    def _():
      pltpu.sync_copy(
          acc_sh.at[pl.ds((NS - 1) * ROWS_PER_TEC, LAST_OUT_ROWS)],
          out_hbm.at[c].at[pl.ds((NS - 1) * ROWS_PER_TEC, LAST_OUT_ROWS)])

  return seg_sum(x_st, src, dst, zrows)


BLK = 1000


def _tc_dense(u_st, x, W, gamma, beta):
  def body(u_ref, x_ref, w_ref, g_ref, b_ref, o_ref):
    dn = (((1,), (1,)), ((), ()))
    acc = lax.dot_general(u_ref[0], w_ref[:, :HALF], dn,
                          preferred_element_type=jnp.float32)
    acc = acc + lax.dot_general(u_ref[1], w_ref[:, HALF:], dn,
                                preferred_element_type=jnp.float32)
    h = jnp.maximum(acc, 0.0) + x_ref[...]
    mean = jnp.mean(h, axis=1, keepdims=True)
    d = h - mean
    var = jnp.mean(d * d, axis=1, keepdims=True)
    o_ref[...] = d * lax.rsqrt(var + 1e-5) * g_ref[...] + b_ref[...]

  return pl.pallas_call(
      body,
      grid=(N_NODES // BLK,),
      in_specs=[
          pl.BlockSpec((2, BLK, HALF), lambda i: (0, i, 0)),
          pl.BlockSpec((BLK, HIDDEN), lambda i: (i, 0)),
          pl.BlockSpec((HIDDEN, HIDDEN), lambda i: (0, 0)),
          pl.BlockSpec((1, HIDDEN), lambda i: (0, 0)),
          pl.BlockSpec((1, HIDDEN), lambda i: (0, 0)),
      ],
      out_specs=pl.BlockSpec((BLK, HIDDEN), lambda i: (i, 0)),
      out_shape=jax.ShapeDtypeStruct((N_NODES, HIDDEN), jnp.float32),
  )(u_st, x, W, gamma, beta)


def kernel(x, edge_index, W, gamma, beta):
  src = edge_index[0].astype(jnp.int32)
  dst = edge_index[1].astype(jnp.int32)
  npad = E_PAD - N_EDGES
  pad_ids = jnp.arange(npad, dtype=jnp.int32)
  src_p = jnp.concatenate([src, pad_ids % N_NODES])
  dst_p = jnp.concatenate([dst, N_NODES + (pad_ids % N_JUNK)])
  x_st = x.reshape(N_NODES, 2, HALF).transpose(1, 0, 2)
  zrows = jnp.zeros((ROWS_PER_TEC, HALF), jnp.float32)
  u_st = _sc_segment_sum(x_st, src_p, dst_p, zrows)
  return _tc_dense(u_st, x, W,
                   gamma.reshape(1, HIDDEN), beta.reshape(1, HIDDEN))


# speedup vs baseline: 7.0526x; 1.6092x over previous
import functools

import jax
import jax.numpy as jnp
from jax import lax
from jax.experimental import pallas as pl
from jax.experimental.pallas import tpu as pltpu
from jax.experimental.pallas import tpu_sc as plsc

N_NODES = 10000
HIDDEN = 256
HALF = 128
N_EDGES = 160000

NS = 16
CH = 128
NBUF = 2
NIB = 4
CPT = 80
EPT = CPT * CH
E_PAD = EPT * NS
N_JUNK = 8
U_ROWS = N_NODES + N_JUNK
ROWS_PER_TEC = 632
LAST_ROWS = U_ROWS - (NS - 1) * ROWS_PER_TEC
LAST_OUT_ROWS = N_NODES - (NS - 1) * ROWS_PER_TEC


def _sc_segment_sum(x_st, edges, zrows):
  mesh = plsc.VectorSubcoreMesh(core_axis_name="c", subcore_axis_name="s")

  @functools.partial(
      pl.kernel,
      out_type=jax.ShapeDtypeStruct((2, N_NODES, HALF), jnp.float32),
      mesh=mesh,
      scratch_types=[
          pltpu.VMEM((NIB, 2, CH), jnp.int32),
          pltpu.VMEM((NBUF, CH, HALF), jnp.float32),
          pltpu.VMEM_SHARED((U_ROWS, HALF), jnp.float32),
          pltpu.SemaphoreType.DMA,
          pltpu.SemaphoreType.DMA,
          pltpu.SemaphoreType.DMA,
          pltpu.SemaphoreType.DMA,
          pltpu.SemaphoreType.DMA,
          pltpu.SemaphoreType.DMA,
          pltpu.SemaphoreType.DMA,
          pltpu.SemaphoreType.DMA,
      ],
  )
  def seg_sum(x_hbm, e_hbm, z_hbm, out_hbm,
              idx_v, rows_v, acc_sh,
              si0, si1, si2, si3, sg0, sg1, ss0, ss1):
    c = lax.axis_index("c")
    s = lax.axis_index("s")
    si = (si0, si1, si2, si3)
    sg = (sg0, sg1)
    ss = (ss0, ss1)

    row0 = s * ROWS_PER_TEC

    @pl.when(s < NS - 1)
    def _():
      pltpu.sync_copy(z_hbm, acc_sh.at[pl.ds(row0, ROWS_PER_TEC)])

    @pl.when(s == NS - 1)
    def _():
      pltpu.sync_copy(z_hbm.at[pl.ds(0, LAST_ROWS)],
                      acc_sh.at[pl.ds((NS - 1) * ROWS_PER_TEC, LAST_ROWS)])

    plsc.subcore_barrier()

    kbase = s * CPT

    def start_idx(j, m):
      pltpu.async_copy(e_hbm.at[kbase + j], idx_v.at[m], si[m])

    def wait_idx(m):
      pltpu.make_async_copy(e_hbm.at[0], idx_v.at[m], si[m]).wait()

    def start_gather(m, b):
      pltpu.async_copy(x_hbm.at[c].at[idx_v.at[m].at[0]], rows_v.at[b],
                       sg[b])

    def wait_gather(b):
      pltpu.make_async_copy(x_hbm.at[c].at[idx_v.at[0].at[0]],
                            rows_v.at[b], sg[b]).wait()

    def start_scatter(m, b):
      pltpu.async_copy(rows_v.at[b], acc_sh.at[idx_v.at[m].at[1]], ss[b],
                       add=True)

    def wait_scatter(b):
      pltpu.make_async_copy(rows_v.at[b], acc_sh.at[idx_v.at[0].at[1]],
                            ss[b]).wait()

    for m in range(NIB):
      start_idx(m, m)

    @pl.loop(0, CPT // NIB)
    def _(q):
      j0 = q * NIB
      for t in range(NIB):
        j = j0 + t
        b = t % NBUF
        m = t
        mp = (t + 2) % NIB

        if t < 2:
          @pl.when(q > 0)
          def _():
            wait_scatter(b)
            start_idx(j + 2, mp)
        else:
          @pl.when(q < CPT // NIB - 1)
          def _():
            wait_scatter(b)
            start_idx(j + 2, mp)

          @pl.when(q == CPT // NIB - 1)
          def _():
            wait_scatter(b)

        wait_idx(m)
        start_gather(m, b)
        wait_gather(b)
        start_scatter(m, b)

    wait_scatter(0)
    wait_scatter(1)

    plsc.subcore_barrier()

    @pl.when(s < NS - 1)
    def _():
      pltpu.sync_copy(acc_sh.at[pl.ds(row0, ROWS_PER_TEC)],
                      out_hbm.at[c].at[pl.ds(row0, ROWS_PER_TEC)])

    @pl.when(s == NS - 1)
    def _():
      pltpu.sync_copy(
          acc_sh.at[pl.ds((NS - 1) * ROWS_PER_TEC, LAST_OUT_ROWS)],
          out_hbm.at[c].at[pl.ds((NS - 1) * ROWS_PER_TEC, LAST_OUT_ROWS)])

  return seg_sum(x_st, edges, zrows)


BLK = 1000


def _tc_dense(u_st, x, W, gamma, beta):
  def body(u_ref, x_ref, w_ref, g_ref, b_ref, o_ref):
    dn = (((1,), (1,)), ((), ()))
    acc = lax.dot_general(u_ref[0], w_ref[:, :HALF], dn,
                          preferred_element_type=jnp.float32)
    acc = acc + lax.dot_general(u_ref[1], w_ref[:, HALF:], dn,
                                preferred_element_type=jnp.float32)
    h = jnp.maximum(acc, 0.0) + x_ref[...]
    mean = jnp.mean(h, axis=1, keepdims=True)
    d = h - mean
    var = jnp.mean(d * d, axis=1, keepdims=True)
    o_ref[...] = d * lax.rsqrt(var + 1e-5) * g_ref[...] + b_ref[...]

  return pl.pallas_call(
      body,
      grid=(N_NODES // BLK,),
      in_specs=[
          pl.BlockSpec((2, BLK, HALF), lambda i: (0, i, 0)),
          pl.BlockSpec((BLK, HIDDEN), lambda i: (i, 0)),
          pl.BlockSpec((HIDDEN, HIDDEN), lambda i: (0, 0)),
          pl.BlockSpec((1, HIDDEN), lambda i: (0, 0)),
          pl.BlockSpec((1, HIDDEN), lambda i: (0, 0)),
      ],
      out_specs=pl.BlockSpec((BLK, HIDDEN), lambda i: (i, 0)),
      out_shape=jax.ShapeDtypeStruct((N_NODES, HIDDEN), jnp.float32),
  )(u_st, x, W, gamma, beta)


def kernel(x, edge_index, W, gamma, beta):
  src = edge_index[0].astype(jnp.int32)
  dst = edge_index[1].astype(jnp.int32)
  npad = E_PAD - N_EDGES
  pad_ids = jnp.arange(npad, dtype=jnp.int32)
  src_p = jnp.concatenate([src, pad_ids % N_NODES]).reshape(NS * CPT, CH)
  dst_p = jnp.concatenate([dst, N_NODES + (pad_ids % N_JUNK)]
                          ).reshape(NS * CPT, CH)
  edges = jnp.stack([src_p, dst_p], axis=1)
  x_st = x.reshape(N_NODES, 2, HALF).transpose(1, 0, 2)
  zrows = jnp.zeros((ROWS_PER_TEC, HALF), jnp.float32)
  u_st = _sc_segment_sum(x_st, edges, zrows)
  return _tc_dense(u_st, x, W,
                   gamma.reshape(1, HIDDEN), beta.reshape(1, HIDDEN))
